# baseline (speedup 1.0000x reference)
import functools

import jax
import jax.numpy as jnp
from jax import lax
from jax.experimental import pallas as pl
from jax.experimental.pallas import tpu as pltpu
from jax.experimental.pallas import tpu_sc as plsc

ALPHA = 0.2
WIN = 80


def _pre_node_body(nf_ref, w_ref, t0_ref, t1_ref, s2_ref):
    out = jnp.dot(nf_ref[...], w_ref[...], preferred_element_type=jnp.float32)
    t0_ref[...] = out[:, :80]
    t1_ref[...] = out[:, 80:160]
    s2_ref[...] = out[:, 160:176]


def _pre_edge_body(ef_ref, b_ref, el_ref):
    el = jnp.dot(ef_ref[...], b_ref[...], preferred_element_type=jnp.float32)
    el_ref[...] = jnp.concatenate([el, ef_ref[...]], axis=1)


def _sc_edge_pass(E, N, src, dst, elf, tab0, tab1, s_dst_tab, z):
    NC, NS = 2, 16
    per_w = E // NS
    n_win = per_w // WIN
    rows_a = ((N // NS) + 7) // 8 * 8
    rows_last = N - (NS - 1) * rows_a

    mesh = plsc.VectorSubcoreMesh(core_axis_name="c", subcore_axis_name="s")

    @functools.partial(
        pl.kernel,
        out_type=jax.ShapeDtypeStruct((NC, N, 112), jnp.float32),
        mesh=mesh,
        compiler_params=pltpu.CompilerParams(use_tc_tiling_on_sc=False),
        scratch_types=[
            pltpu.VMEM_SHARED((N, 112), jnp.float32),
            pltpu.VMEM((2, WIN), jnp.int32),
            pltpu.VMEM((2, WIN), jnp.int32),
            pltpu.VMEM((2, WIN), jnp.int32),
            pltpu.VMEM((2, WIN, 24), jnp.float32),
            pltpu.VMEM((2, WIN, 16), jnp.float32),
            pltpu.VMEM((2, WIN, 80), jnp.float32),
            pltpu.VMEM((2, WIN, 112), jnp.float32),
            pltpu.SemaphoreType.DMA((2,)),
            pltpu.SemaphoreType.DMA((2,)),
            pltpu.SemaphoreType.DMA((2,)),
        ],
    )
    def sc_fn(src_h, dst_h, elf_h, tab0_h, tab1_h, sdst_h,
              z_h, out_a,
              acc,
              src_w, dstG, dstS, ef_w, s2_w, hs_w, prod,
              semL, semG, semS):
        c = lax.axis_index("c")
        s = lax.axis_index("s")
        iota = jnp.arange(16, dtype=jnp.int32)
        head_mask = iota < 4
        den_mask = jnp.logical_and(head_mask, (iota >> 1) == c)

        r0 = s * rows_a

        @pl.when(s < NS - 1)
        def _():
            pltpu.sync_copy(z_h, acc.at[pl.ds(r0, rows_a)])

        @pl.when(s == NS - 1)
        def _():
            pltpu.sync_copy(z_h.at[pl.ds(0, rows_last)],
                            acc.at[pl.ds(r0, rows_last)])

        plsc.subcore_barrier()

        base = s * per_w

        def issue_linear(w, b):
            e0 = base + w * WIN
            pltpu.async_copy(src_h.at[pl.ds(e0, WIN)], src_w.at[b], semL.at[b])
            pltpu.async_copy(dst_h.at[pl.ds(e0, WIN)], dstG.at[b], semL.at[b])
            pltpu.async_copy(elf_h.at[pl.ds(e0, WIN)], ef_w.at[b], semL.at[b])

        def wait_linear(b):
            pltpu.make_async_copy(src_h.at[pl.ds(0, WIN)], src_w.at[b],
                                  semL.at[b]).wait()
            pltpu.make_async_copy(dst_h.at[pl.ds(0, WIN)], dstG.at[b],
                                  semL.at[b]).wait()
            pltpu.make_async_copy(elf_h.at[pl.ds(0, WIN)], ef_w.at[b],
                                  semL.at[b]).wait()

        def issue_gathers(b):
            pltpu.async_copy(sdst_h.at[dstG.at[b]], s2_w.at[b], semG.at[b])

            @pl.when(c == 0)
            def _():
                pltpu.async_copy(tab0_h.at[src_w.at[b]], hs_w.at[b],
                                 semG.at[b])

            @pl.when(c == 1)
            def _():
                pltpu.async_copy(tab1_h.at[src_w.at[b]], hs_w.at[b],
                                 semG.at[b])

        def wait_gathers(b):
            pltpu.make_async_copy(sdst_h.at[dstG.at[b]], s2_w.at[b],
                                  semG.at[b]).wait()
            pltpu.make_async_copy(tab0_h.at[src_w.at[b]], hs_w.at[b],
                                  semG.at[b]).wait()

        def wait_scatter(b):
            pltpu.make_async_copy(prod.at[b], acc.at[dstS.at[b]],
                                  semS.at[b]).wait()

        def body_set(k, bA, bB):
            @pl.when(k <= n_win - 2)
            def _():
                wait_linear(bB)
                issue_gathers(bB)

            @pl.when(k >= 2)
            def _():
                wait_scatter(bA)

            wait_gathers(bA)
            for i in range(WIN // 16):
                dstS[bA, pl.ds(i * 16, 16)] = dstG[bA, pl.ds(i * 16, 16)]

            def edge(e, carry2):
                lg = (hs_w[bA, e, pl.ds(64, 16)] + s2_w[bA, e, :]
                      + ef_w[bA, e, pl.ds(0, 16)])
                lg = jnp.maximum(lg, lg * ALPHA)
                wrow = jnp.where(head_mask, jnp.exp(lg), 0.0)
                prod[bA, e, pl.ds(96, 16)] = jnp.where(den_mask, wrow, 0.0)
                fv = ef_w[bA, e, pl.ds(8, 16)]
                for h2 in range(2):
                    gidx = jnp.full((16,), h2, jnp.int32) + c * 2
                    splat = wrow.at[gidx].get(mode="promise_in_bounds")
                    prod[bA, e, pl.ds(64 + h2 * 16, 16)] = fv * splat
                    for c2 in range(2):
                        col = h2 * 32 + c2 * 16
                        prod[bA, e, pl.ds(col, 16)] = (
                            hs_w[bA, e, pl.ds(col, 16)] * splat)
                return carry2
            lax.fori_loop(0, WIN, edge, 0)

            pltpu.async_copy(prod.at[bA], acc.at[dstS.at[bA]], semS.at[bA],
                             add=True)

            @pl.when(k <= n_win - 3)
            def _():
                issue_linear(k + 2, bA)

        def window(k, carry):
            @pl.when(k % 2 == 0)
            def _():
                body_set(k, 0, 1)

            @pl.when(k % 2 == 1)
            def _():
                body_set(k, 1, 0)
            return carry

        issue_linear(0, 0)
        wait_linear(0)
        issue_gathers(0)
        issue_linear(1, 1)
        lax.fori_loop(0, n_win, window, 0)
        wait_scatter(0 if (n_win - 2) % 2 == 0 else 1)
        wait_scatter(0 if (n_win - 1) % 2 == 0 else 1)
        plsc.subcore_barrier()

        @pl.when(s < NS - 1)
        def _():
            pltpu.sync_copy(acc.at[pl.ds(r0, rows_a)],
                            out_a.at[c, pl.ds(r0, rows_a)])

        @pl.when(s == NS - 1)
        def _():
            pltpu.sync_copy(acc.at[pl.ds(r0, rows_last)],
                            out_a.at[c, pl.ds(r0, rows_last)])

    return sc_fn(src, dst, elf, tab0, tab1, s_dst_tab, z)


def _post_body(ap_ref, we_ref, v_ref, out_ref):
    p0 = ap_ref[0]
    p1 = ap_ref[1]
    na = jnp.concatenate([p0[:, :64], p1[:, :64]], axis=1)
    ea = jnp.concatenate([p0[:, 64:96], p1[:, 64:96]], axis=1)
    dd = p0[:, 96:100] + p1[:, 96:100]
    blk = na.shape[0]
    inv = v_ref[0:1, :] / (dd + 1e-9)
    inv32 = jnp.broadcast_to(inv[:, :, None], (blk, 4, 32)).reshape(blk, 128)
    inv16 = jnp.broadcast_to(inv[:, :, None], (blk, 4, 16)).reshape(blk, 64)
    node_out = na * inv32
    edge_out = jnp.dot(ea * inv16, we_ref[...],
                       preferred_element_type=jnp.float32)
    out_ref[...] = jnp.concatenate([node_out, edge_out], axis=1)


def kernel(node_fts, edge_fts, edges, W_node, W_edge, a_src, a_dst, a_edge,
           node_att_var):
    N, NODE_IN = node_fts.shape
    E, EDGE_IN = edge_fts.shape
    H, _, NODE_OUT = W_node.shape
    EDGE_OUT = W_edge.shape[2]

    W_cat = jnp.transpose(W_node, (1, 0, 2)).reshape(NODE_IN, H * NODE_OUT)
    c_src = jnp.einsum('hfo,ho->fh', W_node, a_src)
    c_dst = jnp.einsum('hfo,ho->fh', W_node, a_dst)
    padn = jnp.zeros((NODE_IN, 12), jnp.float32)
    W_full = jnp.concatenate(
        [W_cat[:, :64], c_src, padn, W_cat[:, 64:], c_src, padn,
         c_dst, padn], axis=1)
    B_edge = jnp.einsum('hfo,ho->fh', W_edge, a_edge)
    B_pad = jnp.concatenate(
        [B_edge, jnp.zeros((EDGE_IN, 4), jnp.float32)], axis=1)
    eyeH = jnp.eye(H, dtype=jnp.float32)
    We_blk = jnp.einsum('hk,hio->hiko', eyeH, W_edge)
    We_blk = We_blk.reshape(H * EDGE_IN, H * EDGE_OUT)
    v = jnp.exp(jnp.clip(node_att_var, -2.0, 2.0))
    v = v / jnp.sum(v)
    src = edges[:, 0]
    dst = edges[:, 1]

    nblk = N // 5
    tab0, tab1, s_dst_tab = pl.pallas_call(
        _pre_node_body,
        grid=(5,),
        in_specs=[
            pl.BlockSpec((nblk, NODE_IN), lambda i: (i, 0)),
            pl.BlockSpec((NODE_IN, 176), lambda i: (0, 0)),
        ],
        out_specs=[
            pl.BlockSpec((nblk, 80), lambda i: (i, 0)),
            pl.BlockSpec((nblk, 80), lambda i: (i, 0)),
            pl.BlockSpec((nblk, 16), lambda i: (i, 0)),
        ],
        out_shape=[
            jax.ShapeDtypeStruct((N, 80), jnp.float32),
            jax.ShapeDtypeStruct((N, 80), jnp.float32),
            jax.ShapeDtypeStruct((N, 16), jnp.float32),
        ],
    )(node_fts, W_full)

    eblk = E // 40
    elf = pl.pallas_call(
        _pre_edge_body,
        grid=(40,),
        in_specs=[
            pl.BlockSpec((eblk, EDGE_IN), lambda i: (i, 0)),
            pl.BlockSpec((EDGE_IN, 8), lambda i: (0, 0)),
        ],
        out_specs=pl.BlockSpec((eblk, 24), lambda i: (i, 0)),
        out_shape=jax.ShapeDtypeStruct((E, 24), jnp.float32),
    )(edge_fts, B_pad)

    rows_a = ((N // 16) + 7) // 8 * 8
    z = jnp.zeros((rows_a, 112), jnp.float32)
    acc_p = _sc_edge_pass(
        E, N, src, dst, elf, tab0, tab1, s_dst_tab, z)

    vmat = jnp.broadcast_to(v[None, :], (8, H))
    out = pl.pallas_call(
        _post_body,
        grid=(5,),
        in_specs=[
            pl.BlockSpec((2, nblk, 112), lambda i: (0, i, 0)),
            pl.BlockSpec((H * EDGE_IN, H * EDGE_OUT), lambda i: (0, 0)),
            pl.BlockSpec((8, H), lambda i: (0, 0)),
        ],
        out_specs=pl.BlockSpec((nblk, 192), lambda i: (i, 0)),
        out_shape=jax.ShapeDtypeStruct((N, 192), jnp.float32),
    )(acc_p, We_blk, vmat)
    return out


# speedup vs baseline: 1.0091x; 1.0091x over previous
import functools

import jax
import jax.numpy as jnp
from jax import lax
from jax.experimental import pallas as pl
from jax.experimental.pallas import tpu as pltpu
from jax.experimental.pallas import tpu_sc as plsc

ALPHA = 0.2
WIN = 80


def _pre_node_body(nf_ref, w_ref, t0_ref, t1_ref, s2_ref):
    out = jnp.dot(nf_ref[...], w_ref[...], preferred_element_type=jnp.float32)
    t0_ref[...] = out[:, :80]
    t1_ref[...] = out[:, 80:160]
    s2_ref[...] = out[:, 160:176]


def _pre_edge_body(ef_ref, b_ref, el_ref):
    el = jnp.dot(ef_ref[...], b_ref[...], preferred_element_type=jnp.float32)
    el_ref[...] = jnp.concatenate([el, ef_ref[...]], axis=1)


def _sc_edge_pass(E, N, src, dst, elf, tab0, tab1, s_dst_tab, z):
    NC, NS = 2, 16
    per_w = E // NS
    n_win = per_w // WIN
    rows_a = ((N // NS) + 7) // 8 * 8
    rows_last = N - (NS - 1) * rows_a

    mesh = plsc.VectorSubcoreMesh(core_axis_name="c", subcore_axis_name="s")

    @functools.partial(
        pl.kernel,
        out_type=jax.ShapeDtypeStruct((NC, N, 112), jnp.float32),
        mesh=mesh,
        compiler_params=pltpu.CompilerParams(use_tc_tiling_on_sc=False),
        scratch_types=[
            pltpu.VMEM_SHARED((N, 112), jnp.float32),
            pltpu.VMEM((2, WIN), jnp.int32),
            pltpu.VMEM((2, WIN), jnp.int32),
            pltpu.VMEM((2, WIN), jnp.int32),
            pltpu.VMEM((2, WIN, 32), jnp.float32),
            pltpu.VMEM((2, WIN, 16), jnp.float32),
            pltpu.VMEM((2, WIN, 80), jnp.float32),
            pltpu.VMEM((2, WIN, 112), jnp.float32),
            pltpu.SemaphoreType.DMA((2,)),
            pltpu.SemaphoreType.DMA((2,)),
            pltpu.SemaphoreType.DMA((2,)),
        ],
    )
    def sc_fn(src_h, dst_h, elf_h, tab0_h, tab1_h, sdst_h,
              z_h, out_a,
              acc,
              src_w, dstG, dstS, ef_w, s2_w, hs_w, prod,
              semL, semG, semS):
        c = lax.axis_index("c")
        s = lax.axis_index("s")
        iota = jnp.arange(16, dtype=jnp.int32)
        head_mask = iota < 4
        den_mask = jnp.logical_and(head_mask, (iota >> 1) == c)

        r0 = s * rows_a

        @pl.when(s < NS - 1)
        def _():
            pltpu.sync_copy(z_h, acc.at[pl.ds(r0, rows_a)])

        @pl.when(s == NS - 1)
        def _():
            pltpu.sync_copy(z_h.at[pl.ds(0, rows_last)],
                            acc.at[pl.ds(r0, rows_last)])

        plsc.subcore_barrier()

        base = s * per_w

        def issue_linear(w, b):
            e0 = base + w * WIN
            pltpu.async_copy(src_h.at[pl.ds(e0, WIN)], src_w.at[b], semL.at[b])
            pltpu.async_copy(dst_h.at[pl.ds(e0, WIN)], dstG.at[b], semL.at[b])
            pltpu.async_copy(elf_h.at[pl.ds(e0, WIN)], ef_w.at[b], semL.at[b])

        def wait_linear(b):
            pltpu.make_async_copy(src_h.at[pl.ds(0, WIN)], src_w.at[b],
                                  semL.at[b]).wait()
            pltpu.make_async_copy(dst_h.at[pl.ds(0, WIN)], dstG.at[b],
                                  semL.at[b]).wait()
            pltpu.make_async_copy(elf_h.at[pl.ds(0, WIN)], ef_w.at[b],
                                  semL.at[b]).wait()

        def issue_gathers(b):
            pltpu.async_copy(sdst_h.at[dstG.at[b]], s2_w.at[b], semG.at[b])

            @pl.when(c == 0)
            def _():
                pltpu.async_copy(tab0_h.at[src_w.at[b]], hs_w.at[b],
                                 semG.at[b])

            @pl.when(c == 1)
            def _():
                pltpu.async_copy(tab1_h.at[src_w.at[b]], hs_w.at[b],
                                 semG.at[b])

        def wait_gathers(b):
            pltpu.make_async_copy(sdst_h.at[dstG.at[b]], s2_w.at[b],
                                  semG.at[b]).wait()
            pltpu.make_async_copy(tab0_h.at[src_w.at[b]], hs_w.at[b],
                                  semG.at[b]).wait()

        def wait_scatter(b):
            pltpu.make_async_copy(prod.at[b], acc.at[dstS.at[b]],
                                  semS.at[b]).wait()

        def body_set(k, bA, bB):
            @pl.when(k <= n_win - 2)
            def _():
                wait_linear(bB)
                issue_gathers(bB)

            @pl.when(k >= 2)
            def _():
                wait_scatter(bA)

            wait_gathers(bA)
            for i in range(WIN // 16):
                dstS[bA, pl.ds(i * 16, 16)] = dstG[bA, pl.ds(i * 16, 16)]

            def edge(e, carry2):
                lg = (hs_w[bA, e, pl.ds(64, 16)] + s2_w[bA, e, :]
                      + ef_w[bA, e, pl.ds(0, 16)])
                lg = jnp.maximum(lg, lg * ALPHA)
                wrow = jnp.where(head_mask, jnp.exp(lg), 0.0)
                prod[bA, e, pl.ds(96, 16)] = jnp.where(den_mask, wrow, 0.0)
                fv = ef_w[bA, e, pl.ds(16, 16)]
                for h2 in range(2):
                    gidx = jnp.full((16,), h2, jnp.int32) + c * 2
                    splat = wrow.at[gidx].get(mode="promise_in_bounds")
                    prod[bA, e, pl.ds(64 + h2 * 16, 16)] = fv * splat
                    for c2 in range(2):
                        col = h2 * 32 + c2 * 16
                        prod[bA, e, pl.ds(col, 16)] = (
                            hs_w[bA, e, pl.ds(col, 16)] * splat)
                return carry2
            lax.fori_loop(0, WIN, edge, 0)

            pltpu.async_copy(prod.at[bA], acc.at[dstS.at[bA]], semS.at[bA],
                             add=True)

            @pl.when(k <= n_win - 3)
            def _():
                issue_linear(k + 2, bA)

        def window(k, carry):
            @pl.when(k % 2 == 0)
            def _():
                body_set(k, 0, 1)

            @pl.when(k % 2 == 1)
            def _():
                body_set(k, 1, 0)
            return carry

        issue_linear(0, 0)
        wait_linear(0)
        issue_gathers(0)
        issue_linear(1, 1)
        lax.fori_loop(0, n_win, window, 0)
        wait_scatter(0 if (n_win - 2) % 2 == 0 else 1)
        wait_scatter(0 if (n_win - 1) % 2 == 0 else 1)
        plsc.subcore_barrier()

        @pl.when(s < NS - 1)
        def _():
            pltpu.sync_copy(acc.at[pl.ds(r0, rows_a)],
                            out_a.at[c, pl.ds(r0, rows_a)])

        @pl.when(s == NS - 1)
        def _():
            pltpu.sync_copy(acc.at[pl.ds(r0, rows_last)],
                            out_a.at[c, pl.ds(r0, rows_last)])

    return sc_fn(src, dst, elf, tab0, tab1, s_dst_tab, z)


def _post_body(ap_ref, we_ref, v_ref, out_ref):
    p0 = ap_ref[0]
    p1 = ap_ref[1]
    na = jnp.concatenate([p0[:, :64], p1[:, :64]], axis=1)
    ea = jnp.concatenate([p0[:, 64:96], p1[:, 64:96]], axis=1)
    dd = p0[:, 96:100] + p1[:, 96:100]
    blk = na.shape[0]
    inv = v_ref[0:1, :] / (dd + 1e-9)
    inv32 = jnp.broadcast_to(inv[:, :, None], (blk, 4, 32)).reshape(blk, 128)
    inv16 = jnp.broadcast_to(inv[:, :, None], (blk, 4, 16)).reshape(blk, 64)
    node_out = na * inv32
    edge_out = jnp.dot(ea * inv16, we_ref[...],
                       preferred_element_type=jnp.float32)
    out_ref[...] = jnp.concatenate([node_out, edge_out], axis=1)


def kernel(node_fts, edge_fts, edges, W_node, W_edge, a_src, a_dst, a_edge,
           node_att_var):
    N, NODE_IN = node_fts.shape
    E, EDGE_IN = edge_fts.shape
    H, _, NODE_OUT = W_node.shape
    EDGE_OUT = W_edge.shape[2]

    W_cat = jnp.transpose(W_node, (1, 0, 2)).reshape(NODE_IN, H * NODE_OUT)
    c_src = jnp.einsum('hfo,ho->fh', W_node, a_src)
    c_dst = jnp.einsum('hfo,ho->fh', W_node, a_dst)
    padn = jnp.zeros((NODE_IN, 12), jnp.float32)
    W_full = jnp.concatenate(
        [W_cat[:, :64], c_src, padn, W_cat[:, 64:], c_src, padn,
         c_dst, padn], axis=1)
    B_edge = jnp.einsum('hfo,ho->fh', W_edge, a_edge)
    B_pad = jnp.concatenate(
        [B_edge, jnp.zeros((EDGE_IN, 12), jnp.float32)], axis=1)
    eyeH = jnp.eye(H, dtype=jnp.float32)
    We_blk = jnp.einsum('hk,hio->hiko', eyeH, W_edge)
    We_blk = We_blk.reshape(H * EDGE_IN, H * EDGE_OUT)
    v = jnp.exp(jnp.clip(node_att_var, -2.0, 2.0))
    v = v / jnp.sum(v)
    src = edges[:, 0]
    dst = edges[:, 1]

    nblk = N // 5
    tab0, tab1, s_dst_tab = pl.pallas_call(
        _pre_node_body,
        grid=(5,),
        in_specs=[
            pl.BlockSpec((nblk, NODE_IN), lambda i: (i, 0)),
            pl.BlockSpec((NODE_IN, 176), lambda i: (0, 0)),
        ],
        out_specs=[
            pl.BlockSpec((nblk, 80), lambda i: (i, 0)),
            pl.BlockSpec((nblk, 80), lambda i: (i, 0)),
            pl.BlockSpec((nblk, 16), lambda i: (i, 0)),
        ],
        out_shape=[
            jax.ShapeDtypeStruct((N, 80), jnp.float32),
            jax.ShapeDtypeStruct((N, 80), jnp.float32),
            jax.ShapeDtypeStruct((N, 16), jnp.float32),
        ],
    )(node_fts, W_full)

    eblk = E // 40
    elf = pl.pallas_call(
        _pre_edge_body,
        grid=(40,),
        in_specs=[
            pl.BlockSpec((eblk, EDGE_IN), lambda i: (i, 0)),
            pl.BlockSpec((EDGE_IN, 16), lambda i: (0, 0)),
        ],
        out_specs=pl.BlockSpec((eblk, 32), lambda i: (i, 0)),
        out_shape=jax.ShapeDtypeStruct((E, 32), jnp.float32),
    )(edge_fts, B_pad)

    rows_a = ((N // 16) + 7) // 8 * 8
    z = jnp.zeros((rows_a, 112), jnp.float32)
    acc_p = _sc_edge_pass(
        E, N, src, dst, elf, tab0, tab1, s_dst_tab, z)

    vmat = jnp.broadcast_to(v[None, :], (8, H))
    out = pl.pallas_call(
        _post_body,
        grid=(5,),
        in_specs=[
            pl.BlockSpec((2, nblk, 112), lambda i: (0, i, 0)),
            pl.BlockSpec((H * EDGE_IN, H * EDGE_OUT), lambda i: (0, 0)),
            pl.BlockSpec((8, H), lambda i: (0, 0)),
        ],
        out_specs=pl.BlockSpec((nblk, 192), lambda i: (i, 0)),
        out_shape=jax.ShapeDtypeStruct((N, 192), jnp.float32),
    )(acc_p, We_blk, vmat)
    return out


# speedup vs baseline: 1.0152x; 1.0061x over previous
import functools

import jax
import jax.numpy as jnp
from jax import lax
from jax.experimental import pallas as pl
from jax.experimental.pallas import tpu as pltpu
from jax.experimental.pallas import tpu_sc as plsc

ALPHA = 0.2
WIN = 80


def _pre_node_body(nf_ref, w_ref, t0_ref, t1_ref, s2_ref):
    out = jnp.dot(nf_ref[...], w_ref[...], preferred_element_type=jnp.float32)
    t0_ref[...] = out[:, :80]
    t1_ref[...] = out[:, 80:160]
    s2_ref[...] = out[:, 160:176]


def _pre_edge_body(ef_ref, b_ref, el_ref):
    el = jnp.dot(ef_ref[...], b_ref[...], preferred_element_type=jnp.float32)
    el_ref[...] = jnp.concatenate([el, ef_ref[...]], axis=1)


def _sc_edge_pass(E, N, src, dst, elf, tab0, tab1, s_dst_tab, z):
    NC, NS = 2, 16
    per_w = E // NS
    n_win = per_w // WIN
    rows_a = ((N // NS) + 7) // 8 * 8
    rows_last = N - (NS - 1) * rows_a

    mesh = plsc.VectorSubcoreMesh(core_axis_name="c", subcore_axis_name="s")

    @functools.partial(
        pl.kernel,
        out_type=jax.ShapeDtypeStruct((NC, N, 112), jnp.float32),
        mesh=mesh,
        compiler_params=pltpu.CompilerParams(use_tc_tiling_on_sc=False),
        scratch_types=[
            pltpu.VMEM_SHARED((N, 112), jnp.float32),
            pltpu.VMEM((2, WIN), jnp.int32),
            pltpu.VMEM((2, WIN), jnp.int32),
            pltpu.VMEM((2, WIN), jnp.int32),
            pltpu.VMEM((2, WIN, 32), jnp.float32),
            pltpu.VMEM((2, WIN, 16), jnp.float32),
            pltpu.VMEM((2, WIN, 80), jnp.float32),
            pltpu.VMEM((2, WIN, 112), jnp.float32),
            pltpu.SemaphoreType.DMA((2,)),
            pltpu.SemaphoreType.DMA((2,)),
            pltpu.SemaphoreType.DMA((2,)),
        ],
    )
    def sc_fn(src_h, dst_h, elf_h, tab0_h, tab1_h, sdst_h,
              z_h, out_a,
              acc,
              src_w, dstG, dstS, ef_w, s2_w, hs_w, prod,
              semL, semG, semS):
        c = lax.axis_index("c")
        s = lax.axis_index("s")
        iota = jnp.arange(16, dtype=jnp.int32)
        head_mask = iota < 4
        den_mask = jnp.logical_and(head_mask, (iota >> 1) == c)

        r0 = s * rows_a

        @pl.when(s < NS - 1)
        def _():
            pltpu.sync_copy(z_h, acc.at[pl.ds(r0, rows_a)])

        @pl.when(s == NS - 1)
        def _():
            pltpu.sync_copy(z_h.at[pl.ds(0, rows_last)],
                            acc.at[pl.ds(r0, rows_last)])

        plsc.subcore_barrier()

        base = s * per_w

        def issue_linear(w, b):
            e0 = base + w * WIN
            pltpu.async_copy(src_h.at[pl.ds(e0, WIN)], src_w.at[b], semL.at[b])
            pltpu.async_copy(dst_h.at[pl.ds(e0, WIN)], dstG.at[b], semL.at[b])
            pltpu.async_copy(elf_h.at[pl.ds(e0, WIN)], ef_w.at[b], semL.at[b])

        def wait_linear(b):
            pltpu.make_async_copy(src_h.at[pl.ds(0, WIN)], src_w.at[b],
                                  semL.at[b]).wait()
            pltpu.make_async_copy(dst_h.at[pl.ds(0, WIN)], dstG.at[b],
                                  semL.at[b]).wait()
            pltpu.make_async_copy(elf_h.at[pl.ds(0, WIN)], ef_w.at[b],
                                  semL.at[b]).wait()

        def issue_gathers(b):
            pltpu.async_copy(sdst_h.at[dstG.at[b]], s2_w.at[b], semG.at[b])

            @pl.when(c == 0)
            def _():
                pltpu.async_copy(tab0_h.at[src_w.at[b]], hs_w.at[b],
                                 semG.at[b])

            @pl.when(c == 1)
            def _():
                pltpu.async_copy(tab1_h.at[src_w.at[b]], hs_w.at[b],
                                 semG.at[b])

        def wait_gathers(b):
            pltpu.make_async_copy(sdst_h.at[dstG.at[b]], s2_w.at[b],
                                  semG.at[b]).wait()
            pltpu.make_async_copy(tab0_h.at[src_w.at[b]], hs_w.at[b],
                                  semG.at[b]).wait()

        def wait_scatter(b):
            pltpu.make_async_copy(prod.at[b], acc.at[dstS.at[b]],
                                  semS.at[b]).wait()

        def body_set(k, bA, bB):
            @pl.when(k <= n_win - 2)
            def _():
                wait_linear(bB)
                issue_gathers(bB)

            @pl.when(k >= 2)
            def _():
                wait_scatter(bA)

            wait_gathers(bA)
            for i in range(WIN // 16):
                dstS[bA, pl.ds(i * 16, 16)] = dstG[bA, pl.ds(i * 16, 16)]

            def edge4(g, carry2):
                for de in range(4):
                    e = g * 4 + de
                    lg = (hs_w[bA, e, pl.ds(64, 16)] + s2_w[bA, e, :]
                          + ef_w[bA, e, pl.ds(0, 16)])
                    lg = jnp.maximum(lg, lg * ALPHA)
                    wrow = jnp.where(head_mask, jnp.exp(lg), 0.0)
                    prod[bA, e, pl.ds(96, 16)] = jnp.where(den_mask, wrow,
                                                           0.0)
                    fv = ef_w[bA, e, pl.ds(16, 16)]
                    for h2 in range(2):
                        gidx = jnp.full((16,), h2, jnp.int32) + c * 2
                        splat = wrow.at[gidx].get(mode="promise_in_bounds")
                        prod[bA, e, pl.ds(64 + h2 * 16, 16)] = fv * splat
                        for c2 in range(2):
                            col = h2 * 32 + c2 * 16
                            prod[bA, e, pl.ds(col, 16)] = (
                                hs_w[bA, e, pl.ds(col, 16)] * splat)
                return carry2
            lax.fori_loop(0, WIN // 4, edge4, 0)

            pltpu.async_copy(prod.at[bA], acc.at[dstS.at[bA]], semS.at[bA],
                             add=True)

            @pl.when(k <= n_win - 3)
            def _():
                issue_linear(k + 2, bA)

        def window(k, carry):
            @pl.when(k % 2 == 0)
            def _():
                body_set(k, 0, 1)

            @pl.when(k % 2 == 1)
            def _():
                body_set(k, 1, 0)
            return carry

        issue_linear(0, 0)
        wait_linear(0)
        issue_gathers(0)
        issue_linear(1, 1)
        lax.fori_loop(0, n_win, window, 0)
        wait_scatter(0 if (n_win - 2) % 2 == 0 else 1)
        wait_scatter(0 if (n_win - 1) % 2 == 0 else 1)
        plsc.subcore_barrier()

        @pl.when(s < NS - 1)
        def _():
            pltpu.sync_copy(acc.at[pl.ds(r0, rows_a)],
                            out_a.at[c, pl.ds(r0, rows_a)])

        @pl.when(s == NS - 1)
        def _():
            pltpu.sync_copy(acc.at[pl.ds(r0, rows_last)],
                            out_a.at[c, pl.ds(r0, rows_last)])

    return sc_fn(src, dst, elf, tab0, tab1, s_dst_tab, z)


def _post_body(ap_ref, we_ref, v_ref, out_ref):
    p0 = ap_ref[0]
    p1 = ap_ref[1]
    na = jnp.concatenate([p0[:, :64], p1[:, :64]], axis=1)
    ea = jnp.concatenate([p0[:, 64:96], p1[:, 64:96]], axis=1)
    dd = p0[:, 96:100] + p1[:, 96:100]
    blk = na.shape[0]
    inv = v_ref[0:1, :] / (dd + 1e-9)
    inv32 = jnp.broadcast_to(inv[:, :, None], (blk, 4, 32)).reshape(blk, 128)
    inv16 = jnp.broadcast_to(inv[:, :, None], (blk, 4, 16)).reshape(blk, 64)
    node_out = na * inv32
    edge_out = jnp.dot(ea * inv16, we_ref[...],
                       preferred_element_type=jnp.float32)
    out_ref[...] = jnp.concatenate([node_out, edge_out], axis=1)


def kernel(node_fts, edge_fts, edges, W_node, W_edge, a_src, a_dst, a_edge,
           node_att_var):
    N, NODE_IN = node_fts.shape
    E, EDGE_IN = edge_fts.shape
    H, _, NODE_OUT = W_node.shape
    EDGE_OUT = W_edge.shape[2]

    W_cat = jnp.transpose(W_node, (1, 0, 2)).reshape(NODE_IN, H * NODE_OUT)
    c_src = jnp.einsum('hfo,ho->fh', W_node, a_src)
    c_dst = jnp.einsum('hfo,ho->fh', W_node, a_dst)
    padn = jnp.zeros((NODE_IN, 12), jnp.float32)
    W_full = jnp.concatenate(
        [W_cat[:, :64], c_src, padn, W_cat[:, 64:], c_src, padn,
         c_dst, padn], axis=1)
    B_edge = jnp.einsum('hfo,ho->fh', W_edge, a_edge)
    B_pad = jnp.concatenate(
        [B_edge, jnp.zeros((EDGE_IN, 12), jnp.float32)], axis=1)
    eyeH = jnp.eye(H, dtype=jnp.float32)
    We_blk = jnp.einsum('hk,hio->hiko', eyeH, W_edge)
    We_blk = We_blk.reshape(H * EDGE_IN, H * EDGE_OUT)
    v = jnp.exp(jnp.clip(node_att_var, -2.0, 2.0))
    v = v / jnp.sum(v)
    src = edges[:, 0]
    dst = edges[:, 1]

    nblk = N // 5
    tab0, tab1, s_dst_tab = pl.pallas_call(
        _pre_node_body,
        grid=(5,),
        in_specs=[
            pl.BlockSpec((nblk, NODE_IN), lambda i: (i, 0)),
            pl.BlockSpec((NODE_IN, 176), lambda i: (0, 0)),
        ],
        out_specs=[
            pl.BlockSpec((nblk, 80), lambda i: (i, 0)),
            pl.BlockSpec((nblk, 80), lambda i: (i, 0)),
            pl.BlockSpec((nblk, 16), lambda i: (i, 0)),
        ],
        out_shape=[
            jax.ShapeDtypeStruct((N, 80), jnp.float32),
            jax.ShapeDtypeStruct((N, 80), jnp.float32),
            jax.ShapeDtypeStruct((N, 16), jnp.float32),
        ],
    )(node_fts, W_full)

    eblk = E // 40
    elf = pl.pallas_call(
        _pre_edge_body,
        grid=(40,),
        in_specs=[
            pl.BlockSpec((eblk, EDGE_IN), lambda i: (i, 0)),
            pl.BlockSpec((EDGE_IN, 16), lambda i: (0, 0)),
        ],
        out_specs=pl.BlockSpec((eblk, 32), lambda i: (i, 0)),
        out_shape=jax.ShapeDtypeStruct((E, 32), jnp.float32),
    )(edge_fts, B_pad)

    rows_a = ((N // 16) + 7) // 8 * 8
    z = jnp.zeros((rows_a, 112), jnp.float32)
    acc_p = _sc_edge_pass(
        E, N, src, dst, elf, tab0, tab1, s_dst_tab, z)

    vmat = jnp.broadcast_to(v[None, :], (8, H))
    out = pl.pallas_call(
        _post_body,
        grid=(5,),
        in_specs=[
            pl.BlockSpec((2, nblk, 112), lambda i: (0, i, 0)),
            pl.BlockSpec((H * EDGE_IN, H * EDGE_OUT), lambda i: (0, 0)),
            pl.BlockSpec((8, H), lambda i: (0, 0)),
        ],
        out_specs=pl.BlockSpec((nblk, 192), lambda i: (i, 0)),
        out_shape=jax.ShapeDtypeStruct((N, 192), jnp.float32),
    )(acc_p, We_blk, vmat)
    return out
